# baseline (speedup 1.0000x reference)
import jax
import jax.numpy as jnp
from jax.experimental import pallas as pl
from jax.experimental.pallas import tpu as pltpu

_BB = 2048
_GPAD = 2560
_TPAD = 32


def _body(t_ref, x_ref, big_ref, sel_ref, o_ref, pk_ref):
    @pl.when(pl.program_id(0) == 0)
    def _():
        m = (big_ref[0:_TPAD, :] > 0.0).astype(jnp.float32)
        pk_ref[...] = jnp.concatenate(
            [m,
             big_ref[_TPAD:2 * _TPAD, :] * m,
             big_ref[2 * _TPAD:3 * _TPAD, :] * m],
            axis=0).astype(jnp.bfloat16)

    gxy = big_ref[96:98, :]
    xdotg = jax.lax.dot(x_ref[...], gxy,
                        preferred_element_type=jnp.float32)
    x0 = x_ref[:, 0:1]
    x1 = x_ref[:, 1:2]
    x2 = x0 * x0 + x1 * x1
    gx = big_ref[96:97, :]
    gy = big_ref[97:98, :]
    g2 = gx * gx + gy * gy
    sq = jnp.maximum(x2 + g2 - 2.0 * xdotg, 0.0)
    w = jnp.exp(sq * -2.0)

    y = jax.lax.dot_general(
        w.astype(jnp.bfloat16), pk_ref[...], (((1,), (1,)), ((), ())),
        preferred_element_type=jnp.float32)

    tt = t_ref[...]
    d_hi = (big_ref[99:100, 0:_TPAD] < tt).astype(jnp.float32)
    d_lo = (big_ref[98:99, 0:_TPAD] < tt).astype(jnp.float32)
    oh = d_hi - d_lo
    z = y * jnp.concatenate([oh, oh, oh], axis=1)
    s = jax.lax.dot(z, sel_ref[...], precision=jax.lax.Precision.HIGHEST,
                    preferred_element_type=jnp.float32)

    den = s[:, 0:1] + 1e-10
    o_ref[:, 0:1] = -(s[:, 1:2] / den)
    o_ref[:, 1:2] = -(s[:, 2:3] / den)


def kernel(t, x, grid_points, t_edges, grid_adjoints, grid_counts):
    B = x.shape[0]
    G = grid_points.shape[0]
    T = grid_counts.shape[0]

    big = jnp.zeros((104, _GPAD), jnp.float32)
    big = big.at[:T, :G].set(grid_counts)
    big = big.at[_TPAD:_TPAD + T, :G].set(grid_adjoints[:, :, 0])
    big = big.at[2 * _TPAD:2 * _TPAD + T, :G].set(grid_adjoints[:, :, 1])
    big = big.at[96, :G].set(grid_points[:, 0])
    big = big.at[97, :G].set(grid_points[:, 1])
    big = big.at[98, :].set(jnp.inf)
    big = big.at[98, : T - 1].set(t_edges[1:T])
    big = big.at[99, :].set(jnp.inf)
    big = big.at[99, 0].set(-jnp.inf)
    big = big.at[99, 1:T].set(t_edges[1:T])

    sel = jnp.zeros((3 * _TPAD, 8), jnp.float32)
    lane = jnp.arange(3 * _TPAD)
    sel = sel.at[lane, lane // _TPAD].set(1.0)

    return pl.pallas_call(
        _body,
        grid=(B // _BB,),
        in_specs=[
            pl.BlockSpec((_BB, 1), lambda i: (i, 0)),
            pl.BlockSpec((_BB, 2), lambda i: (i, 0)),
            pl.BlockSpec((104, _GPAD), lambda i: (0, 0)),
            pl.BlockSpec((3 * _TPAD, 8), lambda i: (0, 0)),
        ],
        out_specs=pl.BlockSpec((_BB, 2), lambda i: (i, 0)),
        out_shape=jax.ShapeDtypeStruct((B, 2), jnp.float32),
        scratch_shapes=[pltpu.VMEM((3 * _TPAD, _GPAD), jnp.bfloat16)],
    )(t, x, big, sel)


# speedup vs baseline: 1.0205x; 1.0205x over previous
import jax
import jax.numpy as jnp
from jax.experimental import pallas as pl
from jax.experimental.pallas import tpu as pltpu

_BB = 2048
_GPAD = 2560
_TPAD = 32


def _body(t_ref, x_ref, big_ref, sel_ref, o_ref, pk_ref):
    @pl.when(pl.program_id(0) == 0)
    def _():
        m = (big_ref[0:_TPAD, :] > 0.0).astype(jnp.float32)
        pk_ref[...] = jnp.concatenate(
            [m,
             big_ref[_TPAD:2 * _TPAD, :] * m,
             big_ref[2 * _TPAD:3 * _TPAD, :] * m], axis=0)

    gxy = big_ref[96:98, :]
    xdotg = jax.lax.dot(x_ref[...], gxy,
                        preferred_element_type=jnp.float32)
    x0 = x_ref[:, 0:1]
    x1 = x_ref[:, 1:2]
    x2 = x0 * x0 + x1 * x1
    gx = big_ref[96:97, :]
    gy = big_ref[97:98, :]
    g2 = gx * gx + gy * gy
    sq = jnp.maximum(x2 + g2 - 2.0 * xdotg, 0.0)
    w = jnp.exp(sq * -2.0)

    y = jax.lax.dot_general(
        w, pk_ref[...], (((1,), (1,)), ((), ())),
        preferred_element_type=jnp.float32)

    tt = t_ref[...]
    d_hi = (big_ref[99:100, 0:_TPAD] < tt).astype(jnp.float32)
    d_lo = (big_ref[98:99, 0:_TPAD] < tt).astype(jnp.float32)
    oh = d_hi - d_lo
    z = y * jnp.concatenate([oh, oh, oh], axis=1)
    s = jax.lax.dot(z, sel_ref[...], precision=jax.lax.Precision.HIGHEST,
                    preferred_element_type=jnp.float32)

    den = s[:, 0:1] + 1e-10
    o_ref[:, 0:1] = -(s[:, 1:2] / den)
    o_ref[:, 1:2] = -(s[:, 2:3] / den)


def kernel(t, x, grid_points, t_edges, grid_adjoints, grid_counts):
    B = x.shape[0]
    G = grid_points.shape[0]
    T = grid_counts.shape[0]

    big = jnp.zeros((104, _GPAD), jnp.float32)
    big = big.at[:T, :G].set(grid_counts)
    big = big.at[_TPAD:_TPAD + T, :G].set(grid_adjoints[:, :, 0])
    big = big.at[2 * _TPAD:2 * _TPAD + T, :G].set(grid_adjoints[:, :, 1])
    big = big.at[96, :G].set(grid_points[:, 0])
    big = big.at[97, :G].set(grid_points[:, 1])
    big = big.at[98, :].set(jnp.inf)
    big = big.at[98, : T - 1].set(t_edges[1:T])
    big = big.at[99, :].set(jnp.inf)
    big = big.at[99, 0].set(-jnp.inf)
    big = big.at[99, 1:T].set(t_edges[1:T])

    sel = jnp.zeros((3 * _TPAD, 8), jnp.float32)
    lane = jnp.arange(3 * _TPAD)
    sel = sel.at[lane, lane // _TPAD].set(1.0)

    return pl.pallas_call(
        _body,
        grid=(B // _BB,),
        in_specs=[
            pl.BlockSpec((_BB, 1), lambda i: (i, 0)),
            pl.BlockSpec((_BB, 2), lambda i: (i, 0)),
            pl.BlockSpec((104, _GPAD), lambda i: (0, 0)),
            pl.BlockSpec((3 * _TPAD, 8), lambda i: (0, 0)),
        ],
        out_specs=pl.BlockSpec((_BB, 2), lambda i: (i, 0)),
        out_shape=jax.ShapeDtypeStruct((B, 2), jnp.float32),
        scratch_shapes=[pltpu.VMEM((3 * _TPAD, _GPAD), jnp.float32)],
    )(t, x, big, sel)
